# baseline (speedup 1.0000x reference)
import jax
import jax.numpy as jnp
from jax.experimental import pallas as pl

_BR = 16384
_N = 16384
_PB = _N // _BR


def _enqueue_body(vecs_ref, idsp_ref, queue_ref, qids_ref, valid_ref,
                  outq_ref, outi_ref, outv_ref):
    i = pl.program_id(0)

    @pl.when(i < _PB)
    def _prefix():
        outq_ref[...] = vecs_ref[...]
        outi_ref[...] = idsp_ref[...]
        outv_ref[...] = jnp.ones_like(outv_ref)

    @pl.when(i >= _PB)
    def _tail():
        outq_ref[...] = queue_ref[...]
        outi_ref[...] = qids_ref[...]
        outv_ref[...] = valid_ref[...]


def kernel(vecs, ids, queue, queue_ids, valid):
    n, d = vecs.shape
    k = queue.shape[0]
    grid = (pl.cdiv(k, _BR),)

    c = 64
    ids2 = ids.astype(queue_ids.dtype).reshape(n // c, c)
    qids2 = queue_ids.reshape(k // c, c)
    valid2 = valid.astype(jnp.uint8).reshape(k // c, c)
    br2 = _BR // c

    def first(i):
        return (jnp.minimum(i, _PB - 1), 0)

    def ident(i):
        return (i, 0)

    out_shape = (
        jax.ShapeDtypeStruct((k, d), queue.dtype),
        jax.ShapeDtypeStruct((k // c, c), queue_ids.dtype),
        jax.ShapeDtypeStruct((k // c, c), jnp.uint8),
    )
    new_q, new_i2, new_v2 = pl.pallas_call(
        _enqueue_body,
        grid=grid,
        in_specs=[
            pl.BlockSpec((_BR, d), first),
            pl.BlockSpec((br2, c), first),
            pl.BlockSpec((_BR, d), ident),
            pl.BlockSpec((br2, c), ident),
            pl.BlockSpec((br2, c), ident),
        ],
        out_specs=(
            pl.BlockSpec((_BR, d), ident),
            pl.BlockSpec((br2, c), ident),
            pl.BlockSpec((br2, c), ident),
        ),
        out_shape=out_shape,
    )(vecs.astype(queue.dtype), ids2, queue, qids2, valid2)
    return (new_q, new_i2.reshape(k), new_v2.reshape(k).astype(valid.dtype))


# speedup vs baseline: 1.0038x; 1.0038x over previous
import jax
import jax.numpy as jnp
from jax.experimental import pallas as pl
from jax.experimental.pallas import tpu as pltpu

_N = 16384
_CR = 4096
_D = 8
_K = 1000000
_PC = _N // _CR
_NFULL = (_K - _N) // _CR
_REM = _K - _N - _NFULL * _CR
_NC = _PC + _NFULL + (1 if _REM else 0)


def _ring_kernel(vecs, ids2, qids2, valid2, queue,
                 outq, outi, outv, bin_, bout, sids, svalid,
                 in_sem, out_sem, small_sem):
    r2, c2 = sids.shape
    pn2 = _N // c2

    cp_ids_pre = pltpu.make_async_copy(ids2, sids.at[pl.ds(0, pn2)],
                                       small_sem.at[0])
    cp_ids_tail = pltpu.make_async_copy(qids2.at[pl.ds(pn2, r2 - pn2)],
                                        sids.at[pl.ds(pn2, r2 - pn2)],
                                        small_sem.at[1])
    cp_val_tail = pltpu.make_async_copy(valid2.at[pl.ds(pn2, r2 - pn2)],
                                        svalid.at[pl.ds(pn2, r2 - pn2)],
                                        small_sem.at[2])
    cp_ids_pre.start()
    cp_ids_tail.start()
    cp_val_tail.start()

    def start_in_static(c):
        b = c % _D
        src = vecs if c < _PC else queue
        pltpu.make_async_copy(src.at[pl.ds(c * _CR, _CR)],
                              bin_.at[b], in_sem.at[b]).start()

    def start_in_traced(c):
        b = jax.lax.rem(c, _D)

        @pl.when(c < _PC + _NFULL)
        def _():
            pltpu.make_async_copy(queue.at[pl.ds(c * _CR, _CR)],
                                  bin_.at[b], in_sem.at[b]).start()

        @pl.when(c == _PC + _NFULL)
        def _():
            pltpu.make_async_copy(queue.at[pl.ds((_PC + _NFULL) * _CR, _REM)],
                                  bin_.at[b, pl.ds(0, _REM)],
                                  in_sem.at[b]).start()

    for j in range(_D):
        start_in_static(j)

    def step(i, carry):
        b = jax.lax.rem(i, _D)
        full = i < _PC + _NFULL

        @pl.when(full)
        def _():
            pltpu.make_async_copy(queue.at[pl.ds(0, _CR)], bin_.at[b],
                                  in_sem.at[b]).wait()

            @pl.when(i >= _D)
            def _():
                pltpu.make_async_copy(bout.at[b], outq.at[pl.ds(0, _CR)],
                                      out_sem.at[b]).wait()

            bout[b] = bin_[b]
            pltpu.make_async_copy(bout.at[b], outq.at[pl.ds(i * _CR, _CR)],
                                  out_sem.at[b]).start()

        @pl.when(jnp.logical_not(full))
        def _():
            pltpu.make_async_copy(queue.at[pl.ds(0, _REM)],
                                  bin_.at[b, pl.ds(0, _REM)],
                                  in_sem.at[b]).wait()

            @pl.when(i >= _D)
            def _():
                pltpu.make_async_copy(bout.at[b], outq.at[pl.ds(0, _CR)],
                                      out_sem.at[b]).wait()

            bout[b, pl.ds(0, _REM)] = bin_[b, pl.ds(0, _REM)]
            pltpu.make_async_copy(bout.at[b, pl.ds(0, _REM)],
                                  outq.at[pl.ds(i * _CR, _REM)],
                                  out_sem.at[b]).start()

        @pl.when(i + _D < _NC)
        def _():
            start_in_traced(i + _D)

        return carry

    jax.lax.fori_loop(0, _NC, step, 0)

    cp_ids_pre.wait()
    cp_ids_tail.wait()
    cp_val_tail.wait()
    svalid[pl.ds(0, pn2)] = jnp.ones((pn2, c2), svalid.dtype)
    cp_ids_out = pltpu.make_async_copy(sids, outi, small_sem.at[0])
    cp_val_out = pltpu.make_async_copy(svalid, outv, small_sem.at[1])
    cp_ids_out.start()
    cp_val_out.start()

    for c in range(_NC - _D, _NC):
        b = c % _D
        if c == _PC + _NFULL:
            pltpu.make_async_copy(bout.at[b, pl.ds(0, _REM)],
                                  outq.at[pl.ds(0, _REM)],
                                  out_sem.at[b]).wait()
        else:
            pltpu.make_async_copy(bout.at[b], outq.at[pl.ds(0, _CR)],
                                  out_sem.at[b]).wait()

    cp_ids_out.wait()
    cp_val_out.wait()


def kernel(vecs, ids, queue, queue_ids, valid):
    n, d = vecs.shape
    k = queue.shape[0]
    c = 64
    ids2 = ids.astype(queue_ids.dtype).reshape(n // c, c)
    qids2 = queue_ids.reshape(k // c, c)
    valid2 = valid.astype(jnp.uint8).reshape(k // c, c)

    out_shape = (
        jax.ShapeDtypeStruct((k, d), queue.dtype),
        jax.ShapeDtypeStruct((k // c, c), queue_ids.dtype),
        jax.ShapeDtypeStruct((k // c, c), jnp.uint8),
    )
    anyspec = pl.BlockSpec(memory_space=pl.ANY)
    new_q, new_i2, new_v2 = pl.pallas_call(
        _ring_kernel,
        out_shape=out_shape,
        in_specs=[anyspec] * 5,
        out_specs=(anyspec, anyspec, anyspec),
        scratch_shapes=[
            pltpu.VMEM((_D, _CR, d), queue.dtype),
            pltpu.VMEM((_D, _CR, d), queue.dtype),
            pltpu.VMEM((k // c, c), queue_ids.dtype),
            pltpu.VMEM((k // c, c), jnp.uint8),
            pltpu.SemaphoreType.DMA((_D,)),
            pltpu.SemaphoreType.DMA((_D,)),
            pltpu.SemaphoreType.DMA((3,)),
        ],
    )(vecs.astype(queue.dtype), ids2, qids2, valid2, queue)
    return (new_q, new_i2.reshape(k), new_v2.reshape(k).astype(valid.dtype))
